# Initial kernel scaffold; baseline (speedup 1.0000x reference)
import jax
import jax.numpy as jnp
from jax.experimental import pallas as pl


def kernel(x, cluster_centers, temperature, cluster_weights, W1, b1, W2, b2):
    raise NotImplementedError("write your pallas kernel here")


# speedup vs baseline: 19.0664x; 19.0664x over previous
import jax
import jax.numpy as jnp
from jax.experimental import pallas as pl
from jax.experimental.pallas import tpu as pltpu

_K = 5


def _fused_kernel(x_ref, cc_ref, t_ref, cw_ref, w1_ref, b1_ref, w2_ref, b2_ref,
                  enc_ref, assign_ref, knn_ref, stats_ref,
                  intra_ref, inter_ref, loss_ref,
                  nrow_ref, acc_ref, *, BR: int):
    i = pl.program_id(0)
    nsteps = pl.num_programs(0)
    B, D = x_ref.shape
    C = cc_ref.shape[0]

    x = x_ref[...]
    cc = cc_ref[...]

    @pl.when(i == 0)
    def _init():
        ones = jnp.ones((1, D), jnp.float32)
        nrow_ref[...] = jax.lax.dot_general(
            ones, x * x, (((1,), (1,)), ((), ())),
            preferred_element_type=jnp.float32)
        acc_ref[...] = jnp.zeros((1, 1), jnp.float32)

    xb = x_ref[pl.ds(i * BR, BR), :]
    n_row = nrow_ref[...]
    nb = jnp.sum(xb * xb, axis=1, keepdims=True)

    g = jax.lax.dot_general(
        xb.astype(jnp.bfloat16), x.astype(jnp.bfloat16),
        (((1,), (1,)), ((), ())), preferred_element_type=jnp.float32)
    s = n_row - 2.0 * g
    rows = i * BR + jax.lax.broadcasted_iota(jnp.int32, (BR, B), 0)
    cols = jax.lax.broadcasted_iota(jnp.int32, (BR, B), 1)
    s = jnp.where(rows == cols, jnp.inf, s)

    mins = []
    for k in range(_K):
        m = jnp.min(s, axis=1)
        mins.append(m.reshape(BR, 1))
        if k < _K - 1:
            s = jnp.where(s == m[:, None], jnp.inf, s)
    d2k = jnp.concatenate(mins, axis=1) + nb
    knn = jnp.sqrt(jnp.maximum(d2k, 1e-12))
    knn_ref[...] = knn

    ones = jnp.ones((1, D), jnp.float32)
    ncc_row = jax.lax.dot_general(
        ones, cc * cc, (((1,), (1,)), ((), ())),
        preferred_element_type=jnp.float32)
    gc = jax.lax.dot_general(
        xb, cc, (((1,), (1,)), ((), ())),
        preferred_element_type=jnp.float32)
    d2c = nb + ncc_row - 2.0 * gc
    dc = jnp.sqrt(jnp.maximum(d2c, 1e-12))
    logits = -dc / t_ref[...]
    mx = jnp.max(logits, axis=1, keepdims=True)
    e = jnp.exp(logits - mx)
    a = e / jnp.sum(e, axis=1, keepdims=True)
    a = a * cw_ref[...]
    assign_ref[...] = a

    lm = jnp.mean(xb, axis=1, keepdims=True)
    var = (nb - D * lm * lm) / (D - 1)
    ls = jnp.sqrt(jnp.maximum(var, 0.0)) + 1e-8
    xmx = jnp.max(xb, axis=1, keepdims=True)
    ex = jnp.exp(xb - xmx)
    se = jnp.sum(ex, axis=1, keepdims=True)
    lse = xmx + jnp.log(se)
    p = ex / se
    ent = lse - jnp.sum(p * xb, axis=1, keepdims=True)
    stats = jnp.concatenate([lm, ls, ent], axis=1)
    stats_ref[...] = stats

    feats = jnp.concatenate([a, knn, stats], axis=1)
    h = jnp.maximum(
        jax.lax.dot_general(feats, w1_ref[...], (((1,), (0,)), ((), ())),
                            preferred_element_type=jnp.float32) + b1_ref[...],
        0.0)
    enc_ref[...] = jax.lax.dot_general(
        h, w2_ref[...], (((1,), (0,)), ((), ())),
        preferred_element_type=jnp.float32) + b2_ref[...]

    acc_ref[...] += jnp.sum(dc * a, keepdims=True)

    @pl.when(i == nsteps - 1)
    def _final():
        ncc_col = jnp.sum(cc * cc, axis=1, keepdims=True)
        gcc = jax.lax.dot_general(cc, cc, (((1,), (1,)), ((), ())),
                                  preferred_element_type=jnp.float32)
        d2cc = ncc_col + ncc_row - 2.0 * gcc
        dcc = jnp.sqrt(jnp.maximum(d2cc, 1e-12))
        ri = jax.lax.broadcasted_iota(jnp.int32, (C, C), 0)
        ci = jax.lax.broadcasted_iota(jnp.int32, (C, C), 1)
        inter = jnp.sum(jnp.where(ri == ci, 0.0, dcc), keepdims=True) / (C * (C - 1))
        intra = acc_ref[...] / (B * C)
        intra_ref[...] = intra
        inter_ref[...] = inter
        loss_ref[...] = intra - 0.1 * inter


def kernel(x, cluster_centers, temperature, cluster_weights, W1, b1, W2, b2):
    B, D = x.shape
    C = cluster_centers.shape[0]
    BR = 256
    nsteps = B // BR

    full = lambda shape: pl.BlockSpec(shape, lambda i: (0, 0))
    blocked = lambda w: pl.BlockSpec((BR, w), lambda i: (i, 0))

    out_shape = [
        jax.ShapeDtypeStruct((B, W2.shape[1]), jnp.float32),
        jax.ShapeDtypeStruct((B, C), jnp.float32),
        jax.ShapeDtypeStruct((B, _K), jnp.float32),
        jax.ShapeDtypeStruct((B, 3), jnp.float32),
        jax.ShapeDtypeStruct((1, 1), jnp.float32),
        jax.ShapeDtypeStruct((1, 1), jnp.float32),
        jax.ShapeDtypeStruct((1, 1), jnp.float32),
    ]
    out_specs = [
        blocked(W2.shape[1]), blocked(C), blocked(_K), blocked(3),
        full((1, 1)), full((1, 1)), full((1, 1)),
    ]
    in_specs = [
        full((B, D)), full((C, D)), full((1, 1)), full((1, C)),
        full(W1.shape), full((1, b1.shape[0])),
        full(W2.shape), full((1, b2.shape[0])),
    ]

    import functools
    enc, assign, knn_d, stats, intra, inter, loss = pl.pallas_call(
        functools.partial(_fused_kernel, BR=BR),
        grid=(nsteps,),
        in_specs=in_specs,
        out_specs=out_specs,
        out_shape=out_shape,
        scratch_shapes=[
            pltpu.VMEM((1, B), jnp.float32),
            pltpu.VMEM((1, 1), jnp.float32),
        ],
    )(x, cluster_centers,
      temperature.reshape(1, 1).astype(jnp.float32),
      cluster_weights.reshape(1, C),
      W1, b1.reshape(1, -1), W2, b2.reshape(1, -1))

    return (enc, assign, knn_d, stats,
            loss.reshape(()), intra.reshape(()), inter.reshape(()))


# speedup vs baseline: 21.6298x; 1.1344x over previous
import functools

import jax
import jax.numpy as jnp
from jax.experimental import pallas as pl
from jax.experimental.pallas import tpu as pltpu

_K = 5
_SB = 32


def _fused_kernel(x_ref, cc_ref, t_ref, cw_ref, w1_ref, b1_ref, w2_ref, b2_ref,
                  enc_ref, assign_ref, knn_ref, stats_ref,
                  intra_ref, inter_ref, loss_ref,
                  nh_ref, xbf_ref, acc_ref, *, BR: int):
    i = pl.program_id(0)
    nsteps = pl.num_programs(0)
    B, D = x_ref.shape
    C = cc_ref.shape[0]

    cc = cc_ref[...]

    @pl.when(i == 0)
    def _init():
        x = x_ref[...]
        half = jnp.full((1, D), 0.5, jnp.float32)
        nh_ref[...] = jax.lax.dot_general(
            half, x * x, (((1,), (1,)), ((), ())),
            preferred_element_type=jnp.float32)
        xbf_ref[...] = x.astype(jnp.bfloat16)
        acc_ref[...] = jnp.zeros((1, 1), jnp.float32)

    xb = x_ref[pl.ds(i * BR, BR), :]
    nh = nh_ref[...]
    nb = jnp.sum(xb * xb, axis=1, keepdims=True)

    g = jax.lax.dot_general(
        xbf_ref[pl.ds(i * BR, BR), :], xbf_ref[...],
        (((1,), (1,)), ((), ())), preferred_element_type=jnp.float32)

    inf = jnp.float32(jnp.inf)
    knn_rows = []
    for sb in range(BR // _SB):
        r0 = sb * _SB
        v_sorted = [jnp.full((_SB, 128), inf, jnp.float32) for _ in range(_K + 1)]
        for c in range(B // 128):
            v = nh[:, c * 128:(c + 1) * 128] - g[r0:r0 + _SB, c * 128:(c + 1) * 128]
            for j in range(_K + 1):
                lo = jnp.minimum(v_sorted[j], v)
                if j < _K:
                    v = jnp.maximum(v_sorted[j], v)
                v_sorted[j] = lo
        outs = []
        for k in range(_K + 1):
            m = jnp.min(v_sorted[0], axis=1, keepdims=True)
            if k > 0:
                outs.append(m)
            if k < _K:
                mask = v_sorted[0] == m
                for j in range(_K):
                    v_sorted[j] = jnp.where(mask, v_sorted[j + 1], v_sorted[j])
                v_sorted[_K] = jnp.where(mask, inf, v_sorted[_K])
        d2k = 2.0 * jnp.concatenate(outs, axis=1) + nb[r0:r0 + _SB, :]
        knn_rows.append(jnp.sqrt(jnp.maximum(d2k, 1e-12)))
    knn = jnp.concatenate(knn_rows, axis=0)
    knn_ref[...] = knn

    ones = jnp.ones((1, D), jnp.float32)
    ncc_row = jax.lax.dot_general(
        ones, cc * cc, (((1,), (1,)), ((), ())),
        preferred_element_type=jnp.float32)
    gc = jax.lax.dot_general(
        xb, cc, (((1,), (1,)), ((), ())),
        preferred_element_type=jnp.float32)
    d2c = nb + ncc_row - 2.0 * gc
    dc = jnp.sqrt(jnp.maximum(d2c, 1e-12))
    logits = -dc / t_ref[...]
    mx = jnp.max(logits, axis=1, keepdims=True)
    e = jnp.exp(logits - mx)
    a = e / jnp.sum(e, axis=1, keepdims=True)
    a = a * cw_ref[...]
    assign_ref[...] = a

    lm = jnp.mean(xb, axis=1, keepdims=True)
    var = (nb - D * lm * lm) / (D - 1)
    ls = jnp.sqrt(jnp.maximum(var, 0.0)) + 1e-8
    xmx = jnp.max(xb, axis=1, keepdims=True)
    ex = jnp.exp(xb - xmx)
    se = jnp.sum(ex, axis=1, keepdims=True)
    lse = xmx + jnp.log(se)
    p = ex / se
    ent = lse - jnp.sum(p * xb, axis=1, keepdims=True)
    stats = jnp.concatenate([lm, ls, ent], axis=1)
    stats_ref[...] = stats

    feats = jnp.concatenate([a, knn, stats], axis=1)
    h = jnp.maximum(
        jax.lax.dot_general(feats, w1_ref[...], (((1,), (0,)), ((), ())),
                            preferred_element_type=jnp.float32) + b1_ref[...],
        0.0)
    enc_ref[...] = jax.lax.dot_general(
        h, w2_ref[...], (((1,), (0,)), ((), ())),
        preferred_element_type=jnp.float32) + b2_ref[...]

    acc_ref[...] += jnp.sum(dc * a, keepdims=True)

    @pl.when(i == nsteps - 1)
    def _final():
        ncc_col = jnp.sum(cc * cc, axis=1, keepdims=True)
        gcc = jax.lax.dot_general(cc, cc, (((1,), (1,)), ((), ())),
                                  preferred_element_type=jnp.float32)
        d2cc = ncc_col + ncc_row - 2.0 * gcc
        dcc = jnp.sqrt(jnp.maximum(d2cc, 1e-12))
        ri = jax.lax.broadcasted_iota(jnp.int32, (C, C), 0)
        ci = jax.lax.broadcasted_iota(jnp.int32, (C, C), 1)
        inter = jnp.sum(jnp.where(ri == ci, 0.0, dcc), keepdims=True) / (C * (C - 1))
        intra = acc_ref[...] / (B * C)
        intra_ref[...] = intra
        inter_ref[...] = inter
        loss_ref[...] = intra - 0.1 * inter


def kernel(x, cluster_centers, temperature, cluster_weights, W1, b1, W2, b2):
    B, D = x.shape
    C = cluster_centers.shape[0]
    BR = 256
    nsteps = B // BR

    full = lambda shape: pl.BlockSpec(shape, lambda i: (0, 0))
    blocked = lambda w: pl.BlockSpec((BR, w), lambda i: (i, 0))

    out_shape = [
        jax.ShapeDtypeStruct((B, W2.shape[1]), jnp.float32),
        jax.ShapeDtypeStruct((B, C), jnp.float32),
        jax.ShapeDtypeStruct((B, _K), jnp.float32),
        jax.ShapeDtypeStruct((B, 3), jnp.float32),
        jax.ShapeDtypeStruct((1, 1), jnp.float32),
        jax.ShapeDtypeStruct((1, 1), jnp.float32),
        jax.ShapeDtypeStruct((1, 1), jnp.float32),
    ]
    out_specs = [
        blocked(W2.shape[1]), blocked(C), blocked(_K), blocked(3),
        full((1, 1)), full((1, 1)), full((1, 1)),
    ]
    in_specs = [
        full((B, D)), full((C, D)), full((1, 1)), full((1, C)),
        full(W1.shape), full((1, b1.shape[0])),
        full(W2.shape), full((1, b2.shape[0])),
    ]

    enc, assign, knn_d, stats, intra, inter, loss = pl.pallas_call(
        functools.partial(_fused_kernel, BR=BR),
        grid=(nsteps,),
        in_specs=in_specs,
        out_specs=out_specs,
        out_shape=out_shape,
        scratch_shapes=[
            pltpu.VMEM((1, B), jnp.float32),
            pltpu.VMEM((B, D), jnp.bfloat16),
            pltpu.VMEM((1, 1), jnp.float32),
        ],
    )(x, cluster_centers,
      temperature.reshape(1, 1).astype(jnp.float32),
      cluster_weights.reshape(1, C),
      W1, b1.reshape(1, -1), W2, b2.reshape(1, -1))

    return (enc, assign, knn_d, stats,
            loss.reshape(()), intra.reshape(()), inter.reshape(()))


# speedup vs baseline: 23.5790x; 1.0901x over previous
import functools

import jax
import jax.numpy as jnp
from jax.experimental import pallas as pl
from jax.experimental.pallas import tpu as pltpu

_K = 5
_SB = 32


def _fused_kernel(x_ref, cc_ref, t_ref, cw_ref, w1_ref, b1_ref, w2_ref, b2_ref,
                  enc_ref, assign_ref, knn_ref, stats_ref,
                  intra_ref, inter_ref, loss_ref,
                  nh_ref, xbf_ref, acc_ref, *, BR: int):
    i = pl.program_id(0)
    nsteps = pl.num_programs(0)
    B, D = x_ref.shape
    C = cc_ref.shape[0]

    cc = cc_ref[...]

    @pl.when(i == 0)
    def _init():
        x = x_ref[...]
        half = jnp.full((1, D), 0.5, jnp.float32)
        nh_ref[...] = jax.lax.dot_general(
            half, x * x, (((1,), (1,)), ((), ())),
            preferred_element_type=jnp.float32)
        xbf_ref[...] = x.astype(jnp.bfloat16)
        acc_ref[...] = jnp.zeros((1, 1), jnp.float32)

    xb = x_ref[pl.ds(i * BR, BR), :]
    nh = nh_ref[...]
    nb = jnp.sum(xb * xb, axis=1, keepdims=True)

    g = jax.lax.dot_general(
        xbf_ref[pl.ds(i * BR, BR), :], xbf_ref[...],
        (((1,), (1,)), ((), ())), preferred_element_type=jnp.float32)

    inf = jnp.float32(jnp.inf)
    knn_rows = []
    for sb in range(BR // _SB):
        r0 = sb * _SB
        v_sorted = [jnp.full((_SB, 128), inf, jnp.float32) for _ in range(_K + 1)]
        for c in range(B // 128):
            v = nh[:, c * 128:(c + 1) * 128] - g[r0:r0 + _SB, c * 128:(c + 1) * 128]
            for j in range(_K + 1):
                lo = jnp.minimum(v_sorted[j], v)
                if j < _K:
                    v = jnp.maximum(v_sorted[j], v)
                v_sorted[j] = lo
        outs = []
        for k in range(_K + 1):
            m = jnp.min(v_sorted[0], axis=1, keepdims=True)
            if k > 0:
                outs.append(m)
            if k < _K:
                mask = v_sorted[0] == m
                for j in range(_K):
                    v_sorted[j] = jnp.where(mask, v_sorted[j + 1], v_sorted[j])
                v_sorted[_K] = jnp.where(mask, inf, v_sorted[_K])
        d2k = 2.0 * jnp.concatenate(outs, axis=1) + nb[r0:r0 + _SB, :]
        knn_rows.append(jnp.sqrt(jnp.maximum(d2k, 1e-12)))
    knn = jnp.concatenate(knn_rows, axis=0)
    knn_ref[...] = knn

    ones = jnp.ones((1, D), jnp.float32)
    ncc_row = jax.lax.dot_general(
        ones, cc * cc, (((1,), (1,)), ((), ())),
        preferred_element_type=jnp.float32)
    gc = jax.lax.dot_general(
        xb, cc, (((1,), (1,)), ((), ())),
        preferred_element_type=jnp.float32)
    d2c = nb + ncc_row - 2.0 * gc
    dc = jnp.sqrt(jnp.maximum(d2c, 1e-12))
    logits = -dc / t_ref[...]
    mx = jnp.max(logits, axis=1, keepdims=True)
    e = jnp.exp(logits - mx)
    a = e / jnp.sum(e, axis=1, keepdims=True)
    a = a * cw_ref[...]
    assign_ref[...] = a

    lm = jnp.mean(xb, axis=1, keepdims=True)
    var = (nb - D * lm * lm) / (D - 1)
    ls = jnp.sqrt(jnp.maximum(var, 0.0)) + 1e-8
    xmx = jnp.max(xb, axis=1, keepdims=True)
    ex = jnp.exp(xb - xmx)
    se = jnp.sum(ex, axis=1, keepdims=True)
    lse = xmx + jnp.log(se)
    p = ex / se
    ent = lse - jnp.sum(p * xb, axis=1, keepdims=True)
    stats = jnp.concatenate([lm, ls, ent], axis=1)
    stats_ref[...] = stats

    feats = jnp.concatenate([a, knn, stats], axis=1)
    h = jnp.maximum(
        jax.lax.dot_general(feats, w1_ref[...], (((1,), (0,)), ((), ())),
                            preferred_element_type=jnp.float32) + b1_ref[...],
        0.0)
    enc_ref[...] = jax.lax.dot_general(
        h, w2_ref[...], (((1,), (0,)), ((), ())),
        preferred_element_type=jnp.float32) + b2_ref[...]

    acc_ref[...] += jnp.sum(dc * a, keepdims=True)

    @pl.when(i == nsteps - 1)
    def _final():
        ncc_col = jnp.sum(cc * cc, axis=1, keepdims=True)
        gcc = jax.lax.dot_general(cc, cc, (((1,), (1,)), ((), ())),
                                  preferred_element_type=jnp.float32)
        d2cc = ncc_col + ncc_row - 2.0 * gcc
        dcc = jnp.sqrt(jnp.maximum(d2cc, 1e-12))
        ri = jax.lax.broadcasted_iota(jnp.int32, (C, C), 0)
        ci = jax.lax.broadcasted_iota(jnp.int32, (C, C), 1)
        inter = jnp.sum(jnp.where(ri == ci, 0.0, dcc), keepdims=True) / (C * (C - 1))
        intra = acc_ref[...] / (B * C)
        intra_ref[...] = intra
        inter_ref[...] = inter
        loss_ref[...] = intra - 0.1 * inter


def kernel(x, cluster_centers, temperature, cluster_weights, W1, b1, W2, b2):
    B, D = x.shape
    C = cluster_centers.shape[0]
    BR = 512
    nsteps = B // BR

    full = lambda shape: pl.BlockSpec(shape, lambda i: (0, 0))
    blocked = lambda w: pl.BlockSpec((BR, w), lambda i: (i, 0))

    out_shape = [
        jax.ShapeDtypeStruct((B, W2.shape[1]), jnp.float32),
        jax.ShapeDtypeStruct((B, C), jnp.float32),
        jax.ShapeDtypeStruct((B, _K), jnp.float32),
        jax.ShapeDtypeStruct((B, 3), jnp.float32),
        jax.ShapeDtypeStruct((1, 1), jnp.float32),
        jax.ShapeDtypeStruct((1, 1), jnp.float32),
        jax.ShapeDtypeStruct((1, 1), jnp.float32),
    ]
    out_specs = [
        blocked(W2.shape[1]), blocked(C), blocked(_K), blocked(3),
        full((1, 1)), full((1, 1)), full((1, 1)),
    ]
    in_specs = [
        full((B, D)), full((C, D)), full((1, 1)), full((1, C)),
        full(W1.shape), full((1, b1.shape[0])),
        full(W2.shape), full((1, b2.shape[0])),
    ]

    enc, assign, knn_d, stats, intra, inter, loss = pl.pallas_call(
        functools.partial(_fused_kernel, BR=BR),
        grid=(nsteps,),
        in_specs=in_specs,
        out_specs=out_specs,
        out_shape=out_shape,
        scratch_shapes=[
            pltpu.VMEM((1, B), jnp.float32),
            pltpu.VMEM((B, D), jnp.bfloat16),
            pltpu.VMEM((1, 1), jnp.float32),
        ],
    )(x, cluster_centers,
      temperature.reshape(1, 1).astype(jnp.float32),
      cluster_weights.reshape(1, C),
      W1, b1.reshape(1, -1), W2, b2.reshape(1, -1))

    return (enc, assign, knn_d, stats,
            loss.reshape(()), intra.reshape(()), inter.reshape(()))


# speedup vs baseline: 26.9689x; 1.1438x over previous
import functools

import jax
import jax.numpy as jnp
from jax.experimental import pallas as pl
from jax.experimental.pallas import tpu as pltpu

_K = 5
_SB = 8


def _fused_kernel(x_ref, cc_ref, t_ref, cw_ref, w1_ref, b1_ref, w2_ref, b2_ref,
                  enc_ref, assign_ref, knn_ref, stats_ref,
                  intra_ref, inter_ref, loss_ref,
                  nh_ref, xbf_ref, acc_ref, *, BR: int):
    i = pl.program_id(0)
    nsteps = pl.num_programs(0)
    B, D = x_ref.shape
    C = cc_ref.shape[0]

    cc = cc_ref[...]

    @pl.when(i == 0)
    def _init():
        x = x_ref[...]
        half = jnp.full((1, D), 0.5, jnp.float32)
        nh_ref[...] = jax.lax.dot_general(
            half, x * x, (((1,), (1,)), ((), ())),
            preferred_element_type=jnp.float32)
        xbf_ref[...] = x.astype(jnp.bfloat16)
        acc_ref[...] = jnp.zeros((1, 1), jnp.float32)

    xb = x_ref[pl.ds(i * BR, BR), :]
    nh = nh_ref[...]
    nb = jnp.sum(xb * xb, axis=1, keepdims=True)

    g = jax.lax.dot_general(
        xbf_ref[pl.ds(i * BR, BR), :], xbf_ref[...],
        (((1,), (1,)), ((), ())), preferred_element_type=jnp.float32)

    inf = jnp.float32(jnp.inf)

    def _vmin(a, b):
        if a is None:
            return b
        if b is None:
            return a
        return jnp.minimum(a, b)

    def _vmax(a, b):
        if a is None or b is None:
            return None
        return jnp.maximum(a, b)

    def _merge_lowest(seq, r):
        if r <= 0:
            return []
        n = len(seq)
        if n == 1:
            return [seq[0]] if seq[0] is not None else []
        half = n // 2
        lo = [_vmin(seq[i], seq[i + half]) for i in range(half)]
        out = _merge_lowest(lo, min(r, half))
        if r > half:
            hi = [_vmax(seq[i], seq[i + half]) for i in range(half)]
            out += _merge_lowest(hi, r - half)
        return out

    def _merge_sorted(A, Bs, r):
        tot = len(A) + len(Bs)
        n = 1
        while n < tot:
            n *= 2
        return _merge_lowest(A + [None] * (n - tot) + Bs[::-1], r)

    def _tree(vals, c0, c1, cap):
        if c1 - c0 == 1:
            return [vals[c0]]
        mid = (c0 + c1) // 2
        L = _tree(vals, c0, mid, cap)
        R = _tree(vals, mid, c1, cap)
        return _merge_sorted(L, R, min(cap, len(L) + len(R)))

    knn_rows = []
    for rg in range(BR // _SB):
        r0 = rg * _SB
        vals = [nh[:, c * 128:(c + 1) * 128] - g[r0:r0 + _SB, c * 128:(c + 1) * 128]
                for c in range(B // 128)]
        vs = _tree(vals, 0, B // 128, _K + 1)
        outs = []
        for k in range(_K + 1):
            m = jnp.min(vs[0], axis=1, keepdims=True)
            if k > 0:
                outs.append(m)
            if k < _K:
                mask = vs[0] == m
                for j in range(_K):
                    vs[j] = jnp.where(mask, vs[j + 1], vs[j])
                vs[_K] = jnp.where(mask, inf, vs[_K])
        d2k = 2.0 * jnp.concatenate(outs, axis=1) + nb[r0:r0 + _SB, :]
        knn_rows.append(jnp.sqrt(jnp.maximum(d2k, 1e-12)))
    knn = jnp.concatenate(knn_rows, axis=0)
    knn_ref[...] = knn

    ones = jnp.ones((1, D), jnp.float32)
    ncc_row = jax.lax.dot_general(
        ones, cc * cc, (((1,), (1,)), ((), ())),
        preferred_element_type=jnp.float32)
    gc = jax.lax.dot_general(
        xb, cc, (((1,), (1,)), ((), ())),
        preferred_element_type=jnp.float32)
    d2c = nb + ncc_row - 2.0 * gc
    dc = jnp.sqrt(jnp.maximum(d2c, 1e-12))
    logits = -dc / t_ref[...]
    mx = jnp.max(logits, axis=1, keepdims=True)
    e = jnp.exp(logits - mx)
    a = e / jnp.sum(e, axis=1, keepdims=True)
    a = a * cw_ref[...]
    assign_ref[...] = a

    lm = jnp.mean(xb, axis=1, keepdims=True)
    var = (nb - D * lm * lm) / (D - 1)
    ls = jnp.sqrt(jnp.maximum(var, 0.0)) + 1e-8
    xmx = jnp.max(xb, axis=1, keepdims=True)
    ex = jnp.exp(xb - xmx)
    se = jnp.sum(ex, axis=1, keepdims=True)
    lse = xmx + jnp.log(se)
    p = ex / se
    ent = lse - jnp.sum(p * xb, axis=1, keepdims=True)
    stats = jnp.concatenate([lm, ls, ent], axis=1)
    stats_ref[...] = stats

    feats = jnp.concatenate([a, knn, stats], axis=1)
    h = jnp.maximum(
        jax.lax.dot_general(feats, w1_ref[...], (((1,), (0,)), ((), ())),
                            preferred_element_type=jnp.float32) + b1_ref[...],
        0.0)
    enc_ref[...] = jax.lax.dot_general(
        h, w2_ref[...], (((1,), (0,)), ((), ())),
        preferred_element_type=jnp.float32) + b2_ref[...]

    acc_ref[...] += jnp.sum(dc * a, keepdims=True)

    @pl.when(i == nsteps - 1)
    def _final():
        ncc_col = jnp.sum(cc * cc, axis=1, keepdims=True)
        gcc = jax.lax.dot_general(cc, cc, (((1,), (1,)), ((), ())),
                                  preferred_element_type=jnp.float32)
        d2cc = ncc_col + ncc_row - 2.0 * gcc
        dcc = jnp.sqrt(jnp.maximum(d2cc, 1e-12))
        ri = jax.lax.broadcasted_iota(jnp.int32, (C, C), 0)
        ci = jax.lax.broadcasted_iota(jnp.int32, (C, C), 1)
        inter = jnp.sum(jnp.where(ri == ci, 0.0, dcc), keepdims=True) / (C * (C - 1))
        intra = acc_ref[...] / (B * C)
        intra_ref[...] = intra
        inter_ref[...] = inter
        loss_ref[...] = intra - 0.1 * inter


def kernel(x, cluster_centers, temperature, cluster_weights, W1, b1, W2, b2):
    B, D = x.shape
    C = cluster_centers.shape[0]
    BR = 512
    nsteps = B // BR

    full = lambda shape: pl.BlockSpec(shape, lambda i: (0, 0))
    blocked = lambda w: pl.BlockSpec((BR, w), lambda i: (i, 0))

    out_shape = [
        jax.ShapeDtypeStruct((B, W2.shape[1]), jnp.float32),
        jax.ShapeDtypeStruct((B, C), jnp.float32),
        jax.ShapeDtypeStruct((B, _K), jnp.float32),
        jax.ShapeDtypeStruct((B, 3), jnp.float32),
        jax.ShapeDtypeStruct((1, 1), jnp.float32),
        jax.ShapeDtypeStruct((1, 1), jnp.float32),
        jax.ShapeDtypeStruct((1, 1), jnp.float32),
    ]
    out_specs = [
        blocked(W2.shape[1]), blocked(C), blocked(_K), blocked(3),
        full((1, 1)), full((1, 1)), full((1, 1)),
    ]
    in_specs = [
        full((B, D)), full((C, D)), full((1, 1)), full((1, C)),
        full(W1.shape), full((1, b1.shape[0])),
        full(W2.shape), full((1, b2.shape[0])),
    ]

    enc, assign, knn_d, stats, intra, inter, loss = pl.pallas_call(
        functools.partial(_fused_kernel, BR=BR),
        grid=(nsteps,),
        in_specs=in_specs,
        out_specs=out_specs,
        out_shape=out_shape,
        scratch_shapes=[
            pltpu.VMEM((1, B), jnp.float32),
            pltpu.VMEM((B, D), jnp.bfloat16),
            pltpu.VMEM((1, 1), jnp.float32),
        ],
    )(x, cluster_centers,
      temperature.reshape(1, 1).astype(jnp.float32),
      cluster_weights.reshape(1, C),
      W1, b1.reshape(1, -1), W2, b2.reshape(1, -1))

    return (enc, assign, knn_d, stats,
            loss.reshape(()), intra.reshape(()), inter.reshape(()))
